# Initial kernel scaffold; baseline (speedup 1.0000x reference)
import jax
import jax.numpy as jnp
from jax.experimental import pallas as pl


def kernel(user_embedding, item_embedding, user, pos, neg, negs):
    raise NotImplementedError("write your pallas kernel here")


# speedup vs baseline: 1.3579x; 1.3579x over previous
import functools

import jax
import jax.numpy as jnp
from jax import lax
from jax.experimental import pallas as pl
from jax.experimental.pallas import tpu as pltpu
from jax.experimental.pallas import tpu_sc as plsc

_N_USER = 1000000
_EMBED = 32
_REGS = 0.01
_MARGIN = 1.0
_BATCH = 16384
_N_NEGS = 50

_NC = 2
_NS = 16
_NW = _NC * _NS
_BPW = _BATCH // _NW
_CB = 16
_NCHUNK = _BPW // _CB
_GSUB = 100
_NSUB = _CB * _N_NEGS // _GSUB
_LUT = 64

_mesh = plsc.VectorSubcoreMesh(
    core_axis_name="c", subcore_axis_name="s", num_cores=_NC, num_subcores=_NS
)


@functools.partial(
    pl.kernel,
    out_type=(
        jax.ShapeDtypeStruct((_NW, 16), jnp.float32),
        jax.ShapeDtypeStruct((_NW, 16), jnp.float32),
    ),
    mesh=_mesh,
    compiler_params=pltpu.CompilerParams(
        needs_layout_passes=False, use_tc_tiling_on_sc=False
    ),
    scratch_types=(
        pltpu.VMEM((_NCHUNK, _CB), jnp.int32),
        pltpu.VMEM((_NCHUNK, _CB), jnp.int32),
        pltpu.VMEM((_NCHUNK, _CB), jnp.int32),
        pltpu.VMEM((_NCHUNK * _NSUB, _GSUB), jnp.int32),
        pltpu.VMEM((_LUT,), jnp.float32),
        pltpu.VMEM((_CB, _EMBED), jnp.float32),
        pltpu.VMEM((_CB, _EMBED), jnp.float32),
        pltpu.VMEM((_CB, _EMBED), jnp.float32),
        pltpu.VMEM((_CB * _N_NEGS, _EMBED), jnp.float32),
        pltpu.VMEM((16,), jnp.float32),
        pltpu.VMEM((16,), jnp.float32),
        pltpu.SemaphoreType.DMA,
    ),
)
def _disc(user_emb, item_emb, user2, pos2, neg2, negs2, lut, out_h, out_r,
          uidx, pidx, nidx, gidx, lutv, urows, prows, nrows, grows,
          obh, obr, sem):
    wid = lax.axis_index("s") * _NC + lax.axis_index("c")

    pltpu.sync_copy(user2.at[pl.ds(wid * _NCHUNK, _NCHUNK)], uidx)
    pltpu.sync_copy(pos2.at[pl.ds(wid * _NCHUNK, _NCHUNK)], pidx)
    pltpu.sync_copy(neg2.at[pl.ds(wid * _NCHUNK, _NCHUNK)], nidx)
    pltpu.sync_copy(negs2.at[pl.ds(wid * _NCHUNK * _NSUB, _NCHUNK * _NSUB)], gidx)
    pltpu.sync_copy(lut, lutv)

    iota = lax.iota(jnp.int32, 16)
    zero = jnp.zeros((16,), jnp.float32)
    grow0 = iota * _N_NEGS
    cols = [jnp.full((16,), d, jnp.int32) for d in range(_EMBED)]

    def chunk_body(c, carry):
        hacc, racc = carry
        cps = [
            pltpu.async_copy(user_emb.at[uidx.at[c]], urows, sem),
            pltpu.async_copy(item_emb.at[pidx.at[c]], prows, sem),
            pltpu.async_copy(item_emb.at[nidx.at[c]], nrows, sem),
        ]
        for k in range(_NSUB):
            cps.append(
                pltpu.async_copy(
                    item_emb.at[gidx.at[c * _NSUB + k]],
                    grows.at[pl.ds(k * _GSUB, _GSUB)],
                    sem,
                )
            )
        for cp in cps:
            cp.wait()

        u_ds = [plsc.load_gather(urows, [iota, cols[d]]) for d in range(_EMBED)]
        u2 = zero
        for d in range(_EMBED):
            u2 = u2 + u_ds[d] * u_ds[d]

        sp = zero
        dp = zero
        sn = zero
        dn = zero
        for d in range(_EMBED):
            xp = plsc.load_gather(prows, [iota, cols[d]])
            sp = sp + xp * xp
            dp = dp + u_ds[d] * xp
            xn = plsc.load_gather(nrows, [iota, cols[d]])
            sn = sn + xn * xn
            dn = dn + u_ds[d] * xn
        pos_d = u2 + sp - 2.0 * dp
        pn_diff = (sp - sn) - 2.0 * (dp - dn)
        racc = racc + u2 + sp + sn

        def j_body(j, jc):
            cnt, rc = jc
            rows = grow0 + j
            s = zero
            dt = zero
            for d in range(_EMBED):
                x = plsc.load_gather(grows, [rows, cols[d]])
                s = s + x * x
                dt = dt + u_ds[d] * x
            marg = (sp - s) - 2.0 * (dp - dt) + _MARGIN
            cnt = cnt + jnp.where(marg > 0.0, jnp.ones((16,), jnp.int32),
                                  jnp.zeros((16,), jnp.int32))
            return (cnt, rc + s)

        cnt, racc = lax.fori_loop(
            0, _N_NEGS, j_body, (jnp.zeros((16,), jnp.int32), racc)
        )
        logv = plsc.load_gather(lutv, [cnt])
        hacc = hacc + logv * jnp.maximum(_MARGIN + pn_diff, 0.0)
        return (hacc, racc)

    hacc, racc = lax.fori_loop(0, _NCHUNK, chunk_body, (zero, zero))

    obh[...] = hacc
    obr[...] = racc
    pltpu.sync_copy(obh, out_h.at[wid])
    pltpu.sync_copy(obr, out_r.at[wid])


def kernel(user_embedding, item_embedding, user, pos, neg, negs):
    user2 = user.astype(jnp.int32).reshape(_BATCH // _CB, _CB)
    pos2 = pos.astype(jnp.int32).reshape(_BATCH // _CB, _CB)
    neg2 = neg.astype(jnp.int32).reshape(_BATCH // _CB, _CB)
    negs2 = negs.astype(jnp.int32).reshape(_BATCH * _N_NEGS // _GSUB, _GSUB)
    lut = jnp.log(
        jnp.arange(_LUT, dtype=jnp.float32) * (_N_USER / _N_NEGS) + 1.0
    )
    out_h, out_r = _disc(
        user_embedding, item_embedding, user2, pos2, neg2, negs2, lut
    )
    hinge_loss = jnp.sum(out_h) * (1.0 / _BATCH)
    reg_loss = _REGS * 0.5 * jnp.sum(out_r)
    return (hinge_loss, reg_loss)


# speedup vs baseline: 2.5416x; 1.8716x over previous
import functools

import jax
import jax.numpy as jnp
from jax import lax
from jax.experimental import pallas as pl
from jax.experimental.pallas import tpu as pltpu
from jax.experimental.pallas import tpu_sc as plsc

_N_USER = 1000000
_EMBED = 32
_REGS = 0.01
_MARGIN = 1.0
_BATCH = 16384
_N_NEGS = 50

_NC = 2
_NS = 16
_NW = _NC * _NS
_BPW = _BATCH // _NW
_CB = 16
_NCHUNK = _BPW // _CB
_GSUB = 80
_NSUB = _CB * _N_NEGS // _GSUB
_LUT = 64

_mesh = plsc.VectorSubcoreMesh(
    core_axis_name="c", subcore_axis_name="s", num_cores=_NC, num_subcores=_NS
)


@functools.partial(
    pl.kernel,
    out_type=(
        jax.ShapeDtypeStruct((_NW, 16), jnp.float32),
        jax.ShapeDtypeStruct((_NW, 16), jnp.float32),
    ),
    mesh=_mesh,
    compiler_params=pltpu.CompilerParams(
        needs_layout_passes=False, use_tc_tiling_on_sc=False
    ),
    scratch_types=(
        pltpu.VMEM((_BPW,), jnp.int32),
        pltpu.VMEM((_BPW,), jnp.int32),
        pltpu.VMEM((_BPW * _N_NEGS,), jnp.int32),
        pltpu.VMEM((_LUT,), jnp.float32),
        pltpu.VMEM((_CB, _EMBED), jnp.float32),
        pltpu.VMEM((_CB, _EMBED), jnp.float32),
        pltpu.VMEM((_CB, _EMBED), jnp.float32),
        pltpu.VMEM((_CB * _N_NEGS, _EMBED), jnp.float32),
        pltpu.VMEM((_CB, _EMBED), jnp.float32),
        pltpu.VMEM((_CB, _EMBED), jnp.float32),
        pltpu.VMEM((_CB, _EMBED), jnp.float32),
        pltpu.VMEM((_CB * _N_NEGS, _EMBED), jnp.float32),
        pltpu.VMEM((16,), jnp.float32),
        pltpu.VMEM((16,), jnp.float32),
        pltpu.SemaphoreType.DMA,
        pltpu.SemaphoreType.DMA,
    ),
)
def _disc(u_e, item_emb, pos1, neg1, negs1, lut, out_h, out_r,
          pidx, nidx, gidx, lutv, urowsA, prowsA, nrowsA, growsA,
          urowsB, prowsB, nrowsB, growsB, obh, obr, semA, semB):
    wid = lax.axis_index("s") * _NC + lax.axis_index("c")

    pltpu.sync_copy(pos1.at[pl.ds(wid * _BPW, _BPW)], pidx)
    pltpu.sync_copy(neg1.at[pl.ds(wid * _BPW, _BPW)], nidx)
    pltpu.sync_copy(negs1.at[pl.ds(wid * _BPW * _N_NEGS, _BPW * _N_NEGS)], gidx)
    pltpu.sync_copy(lut, lutv)

    iota = lax.iota(jnp.int32, 16)
    zero = jnp.zeros((16,), jnp.float32)
    grow0 = iota * _N_NEGS
    cols = [jnp.full((16,), d, jnp.int32) for d in range(_EMBED)]

    ubase = wid * _BPW

    def fire(c, urows, prows, nrows, grows, sem):
        pltpu.async_copy(u_e.at[pl.ds(ubase + c * _CB, _CB)], urows, sem)
        pltpu.async_copy(item_emb.at[pidx.at[pl.ds(c * _CB, _CB)]], prows, sem)
        pltpu.async_copy(item_emb.at[nidx.at[pl.ds(c * _CB, _CB)]], nrows, sem)
        for k in range(_NSUB):
            pltpu.async_copy(
                item_emb.at[gidx.at[pl.ds(c * _CB * _N_NEGS + k * _GSUB, _GSUB)]],
                grows.at[pl.ds(k * _GSUB, _GSUB)],
                sem,
            )

    def drain(c, urows, prows, nrows, grows, sem):
        pltpu.make_async_copy(u_e.at[pl.ds(ubase + c * _CB, _CB)], urows, sem).wait()
        pltpu.make_async_copy(item_emb.at[pidx.at[pl.ds(c * _CB, _CB)]], prows, sem).wait()
        pltpu.make_async_copy(item_emb.at[nidx.at[pl.ds(c * _CB, _CB)]], nrows, sem).wait()
        for k in range(_NSUB):
            pltpu.make_async_copy(
                item_emb.at[gidx.at[pl.ds(c * _CB * _N_NEGS + k * _GSUB, _GSUB)]],
                grows.at[pl.ds(k * _GSUB, _GSUB)],
                sem,
            ).wait()

    def compute(carry, urows, prows, nrows, grows):
        hacc, racc = carry
        u_ds = [plsc.load_gather(urows, [iota, cols[d]]) for d in range(_EMBED)]
        u2 = zero
        for d in range(_EMBED):
            u2 = u2 + u_ds[d] * u_ds[d]

        sp = zero
        dp = zero
        sn = zero
        dn = zero
        for d in range(_EMBED):
            xp = plsc.load_gather(prows, [iota, cols[d]])
            sp = sp + xp * xp
            dp = dp + u_ds[d] * xp
            xn = plsc.load_gather(nrows, [iota, cols[d]])
            sn = sn + xn * xn
            dn = dn + u_ds[d] * xn
        pn_diff = (sp - sn) - 2.0 * (dp - dn)
        racc = racc + u2 + sp + sn
        thr = sp - 2.0 * dp + _MARGIN
        hterm = jnp.maximum(_MARGIN + pn_diff, 0.0)

        for b in range(_CB):
            u0 = urows[b, pl.ds(0, 16)]
            u1 = urows[b, pl.ds(16, 16)]
            thr_b = thr[b]

            def j_body(j2, jc):
                cntf, rc = jc
                for u in range(2):
                    row = b * _N_NEGS + j2 * 2 + u
                    x0 = grows[row, pl.ds(0, 16)]
                    x1 = grows[row, pl.ds(16, 16)]
                    sq = x0 * x0 + x1 * x1
                    dotv = u0 * x0 + u1 * x1
                    s = jnp.sum(sq)
                    dt = jnp.sum(dotv)
                    c = jnp.where(thr_b - s + 2.0 * dt > 0.0, 1.0, 0.0)
                    cntf = cntf + c
                    rc = rc + sq
                return (cntf, rc)

            cntf, racc = lax.fori_loop(0, _N_NEGS // 2, j_body, (0.0, racc))
            idxv = jnp.full((16,), 0, jnp.int32) + cntf.astype(jnp.int32)
            lv = plsc.load_gather(lutv, [idxv])
            hacc = hacc + lv * (hterm[b] * (1.0 / 16.0))
        return (hacc, racc)

    fire(0, urowsA, prowsA, nrowsA, growsA, semA)

    def pair_body(i, carry):
        c0 = i * 2
        fire(c0 + 1, urowsB, prowsB, nrowsB, growsB, semB)
        drain(c0, urowsA, prowsA, nrowsA, growsA, semA)
        carry = compute(carry, urowsA, prowsA, nrowsA, growsA)

        @pl.when(i < _NCHUNK // 2 - 1)
        def _():
            fire(c0 + 2, urowsA, prowsA, nrowsA, growsA, semA)

        drain(c0 + 1, urowsB, prowsB, nrowsB, growsB, semB)
        carry = compute(carry, urowsB, prowsB, nrowsB, growsB)
        return carry

    hacc, racc = lax.fori_loop(0, _NCHUNK // 2, pair_body, (zero, zero))

    obh[...] = hacc
    obr[...] = racc
    pltpu.sync_copy(obh, out_h.at[wid])
    pltpu.sync_copy(obr, out_r.at[wid])


def kernel(user_embedding, item_embedding, user, pos, neg, negs):
    user1 = user.astype(jnp.int32)
    pos1 = pos.astype(jnp.int32)
    neg1 = neg.astype(jnp.int32)
    negs1 = negs.astype(jnp.int32).reshape(_BATCH * _N_NEGS)
    lut = jnp.log(
        jnp.arange(_LUT, dtype=jnp.float32) * (_N_USER / _N_NEGS) + 1.0
    )
    u_e = jnp.take(user_embedding, user1, axis=0)
    out_h, out_r = _disc(u_e, item_embedding, pos1, neg1, negs1, lut)
    hinge_loss = jnp.sum(out_h) * (1.0 / _BATCH)
    reg_loss = _REGS * 0.5 * jnp.sum(out_r)
    return (hinge_loss, reg_loss)


# speedup vs baseline: 2.5459x; 1.0017x over previous
import functools

import jax
import jax.numpy as jnp
from jax import lax
from jax.experimental import pallas as pl
from jax.experimental.pallas import tpu as pltpu
from jax.experimental.pallas import tpu_sc as plsc

_N_USER = 1000000
_EMBED = 32
_REGS = 0.01
_MARGIN = 1.0
_BATCH = 16384
_N_NEGS = 50

_NC = 2
_NS = 16
_NW = _NC * _NS
_BPW = _BATCH // _NW
_CB = 16
_NCHUNK = _BPW // _CB
_GSUB = 80
_NSUB = _CB * _N_NEGS // _GSUB
_LUT = 64

_mesh = plsc.VectorSubcoreMesh(
    core_axis_name="c", subcore_axis_name="s", num_cores=_NC, num_subcores=_NS
)


@functools.partial(
    pl.kernel,
    out_type=(
        jax.ShapeDtypeStruct((_NW, 16), jnp.float32),
        jax.ShapeDtypeStruct((_NW, 16), jnp.float32),
    ),
    mesh=_mesh,
    compiler_params=pltpu.CompilerParams(
        needs_layout_passes=False, use_tc_tiling_on_sc=False
    ),
    scratch_types=(
        pltpu.VMEM((_BPW,), jnp.int32),
        pltpu.VMEM((_BPW,), jnp.int32),
        pltpu.VMEM((_BPW * _N_NEGS,), jnp.int32),
        pltpu.VMEM((_LUT,), jnp.float32),
        pltpu.VMEM((_CB, _EMBED), jnp.float32),
        pltpu.VMEM((_CB, _EMBED), jnp.float32),
        pltpu.VMEM((_CB, _EMBED), jnp.float32),
        pltpu.VMEM((_CB * _N_NEGS, _EMBED), jnp.float32),
        pltpu.VMEM((_CB, _EMBED), jnp.float32),
        pltpu.VMEM((_CB, _EMBED), jnp.float32),
        pltpu.VMEM((_CB, _EMBED), jnp.float32),
        pltpu.VMEM((_CB * _N_NEGS, _EMBED), jnp.float32),
        pltpu.VMEM((16,), jnp.float32),
        pltpu.VMEM((16,), jnp.float32),
        pltpu.SemaphoreType.DMA,
        pltpu.SemaphoreType.DMA,
    ),
)
def _disc(u_e, item_emb, pos1, neg1, negs1, lut, out_h, out_r,
          pidx, nidx, gidx, lutv, urowsA, prowsA, nrowsA, growsA,
          urowsB, prowsB, nrowsB, growsB, obh, obr, semA, semB):
    wid = lax.axis_index("s") * _NC + lax.axis_index("c")

    pltpu.sync_copy(pos1.at[pl.ds(wid * _BPW, _BPW)], pidx)
    pltpu.sync_copy(neg1.at[pl.ds(wid * _BPW, _BPW)], nidx)
    pltpu.sync_copy(negs1.at[pl.ds(wid * _BPW * _N_NEGS, _BPW * _N_NEGS)], gidx)
    pltpu.sync_copy(lut, lutv)

    iota = lax.iota(jnp.int32, 16)
    zero = jnp.zeros((16,), jnp.float32)
    cols = [jnp.full((16,), d, jnp.int32) for d in range(_EMBED)]

    ubase = wid * _BPW

    def fire(c, urows, prows, nrows, grows, sem):
        pltpu.async_copy(u_e.at[pl.ds(ubase + c * _CB, _CB)], urows, sem)
        pltpu.async_copy(item_emb.at[pidx.at[pl.ds(c * _CB, _CB)]], prows, sem)
        pltpu.async_copy(item_emb.at[nidx.at[pl.ds(c * _CB, _CB)]], nrows, sem)
        for k in range(_NSUB):
            pltpu.async_copy(
                item_emb.at[gidx.at[pl.ds(c * _CB * _N_NEGS + k * _GSUB, _GSUB)]],
                grows.at[pl.ds(k * _GSUB, _GSUB)],
                sem,
            )

    def drain(c, urows, prows, nrows, grows, sem):
        pltpu.make_async_copy(u_e.at[pl.ds(ubase + c * _CB, _CB)], urows, sem).wait()
        pltpu.make_async_copy(item_emb.at[pidx.at[pl.ds(c * _CB, _CB)]], prows, sem).wait()
        pltpu.make_async_copy(item_emb.at[nidx.at[pl.ds(c * _CB, _CB)]], nrows, sem).wait()
        for k in range(_NSUB):
            pltpu.make_async_copy(
                item_emb.at[gidx.at[pl.ds(c * _CB * _N_NEGS + k * _GSUB, _GSUB)]],
                grows.at[pl.ds(k * _GSUB, _GSUB)],
                sem,
            ).wait()

    def compute(carry, urows, prows, nrows, grows):
        hacc, racc = carry
        u_ds = [plsc.load_gather(urows, [iota, cols[d]]) for d in range(_EMBED)]
        u2 = zero
        for d in range(_EMBED):
            u2 = u2 + u_ds[d] * u_ds[d]

        sp = zero
        dp = zero
        sn = zero
        dn = zero
        for d in range(_EMBED):
            xp = plsc.load_gather(prows, [iota, cols[d]])
            sp = sp + xp * xp
            dp = dp + u_ds[d] * xp
            xn = plsc.load_gather(nrows, [iota, cols[d]])
            sn = sn + xn * xn
            dn = dn + u_ds[d] * xn
        pn_diff = (sp - sn) - 2.0 * (dp - dn)
        racc = racc + u2 + sp + sn
        thr = sp - 2.0 * dp + _MARGIN
        hterm = jnp.maximum(_MARGIN + pn_diff, 0.0)

        for b in range(_CB):
            u0 = urows[b, pl.ds(0, 16)]
            u1 = urows[b, pl.ds(16, 16)]
            thr_b = thr[b]

            def j_body(j2, jc):
                cntf, rc = jc
                for u in range(2):
                    row = b * _N_NEGS + j2 * 2 + u
                    x0 = grows[row, pl.ds(0, 16)]
                    x1 = grows[row, pl.ds(16, 16)]
                    sq = x0 * x0 + x1 * x1
                    dotv = u0 * x0 + u1 * x1
                    s = jnp.sum(sq)
                    dt = jnp.sum(dotv)
                    c = jnp.where(thr_b - s + 2.0 * dt > 0.0, 1.0, 0.0)
                    cntf = cntf + c
                    rc = rc + sq
                return (cntf, rc)

            cntf, racc = lax.fori_loop(0, _N_NEGS // 2, j_body, (0.0, racc))
            idxv = jnp.full((16,), 0, jnp.int32) + cntf.astype(jnp.int32)
            lv = plsc.load_gather(lutv, [idxv])
            hacc = hacc + lv * (hterm[b] * (1.0 / 16.0))
        return (hacc, racc)

    fire(0, urowsA, prowsA, nrowsA, growsA, semA)

    def pair_body(i, carry):
        c0 = i * 2
        fire(c0 + 1, urowsB, prowsB, nrowsB, growsB, semB)
        drain(c0, urowsA, prowsA, nrowsA, growsA, semA)
        carry = compute(carry, urowsA, prowsA, nrowsA, growsA)

        @pl.when(i < _NCHUNK // 2 - 1)
        def _():
            fire(c0 + 2, urowsA, prowsA, nrowsA, growsA, semA)

        drain(c0 + 1, urowsB, prowsB, nrowsB, growsB, semB)
        carry = compute(carry, urowsB, prowsB, nrowsB, growsB)
        return carry

    hacc, racc = lax.fori_loop(0, _NCHUNK // 2, pair_body, (zero, zero))

    obh[...] = hacc
    obr[...] = racc
    pltpu.sync_copy(obh, out_h.at[wid])
    pltpu.sync_copy(obr, out_r.at[wid])


def kernel(user_embedding, item_embedding, user, pos, neg, negs):
    user1 = user.astype(jnp.int32)
    pos1 = pos.astype(jnp.int32)
    neg1 = neg.astype(jnp.int32)
    negs1 = negs.astype(jnp.int32).reshape(_BATCH * _N_NEGS)
    lut = jnp.log(
        jnp.arange(_LUT, dtype=jnp.float32) * (_N_USER / _N_NEGS) + 1.0
    )
    u_e = jnp.take(user_embedding, user1, axis=0)
    out_h, out_r = _disc(u_e, item_embedding, pos1, neg1, negs1, lut)
    hinge_loss = jnp.sum(out_h) * (1.0 / _BATCH)
    reg_loss = _REGS * 0.5 * jnp.sum(out_r)
    return (hinge_loss, reg_loss)
